# Initial kernel scaffold; baseline (speedup 1.0000x reference)
import jax
import jax.numpy as jnp
from jax.experimental import pallas as pl


def kernel(ray_o, light_probe_pos):
    raise NotImplementedError("write your pallas kernel here")


# speedup vs baseline: 2.6452x; 2.6452x over previous
import functools
import math

import jax
import jax.numpy as jnp
from jax.experimental import pallas as pl

K_C = 16
EPS_A = 1e-5


def _acos(x):
    x = jnp.clip(x, -1.0, 1.0)
    ax = jnp.abs(x)
    p = jnp.float32(-0.0012624911)
    for c in (0.0066700901, -0.0170881256, 0.0308918810, -0.0501743046,
              0.0889789874, -0.2145988016, 1.5707963050):
        p = p * ax + jnp.float32(c)
    r = jnp.sqrt(jnp.maximum(1.0 - ax, 0.0)) * p
    return jnp.where(x < 0.0, jnp.float32(math.pi) - r, r)


def _knn_body(qx_ref, qy_ref, qz_ref, px_ref, py_ref, pz_ref,
              rdx_ref, rdy_ref, rdz_ref, d_ref, az_ref, el_ref, *, P):
    qx = qx_ref[...]
    qy = qy_ref[...]
    qz = qz_ref[...]
    px = px_ref[...]
    py = py_ref[...]
    pz = pz_ref[...]

    dx = qx - px
    dy = qy - py
    dz = qz - pz
    d2 = dx * dx + dy * dy + dz * dz

    bQ = d2.shape[0]
    col = jax.lax.broadcasted_iota(jnp.int32, (bQ, P), 1)
    big = jnp.float32(jnp.inf)

    ds, pxs, pys, pzs = [], [], [], []
    for _ in range(K_C):
        m = jnp.min(d2, axis=1, keepdims=True)
        sel0 = d2 <= m
        idx = jnp.min(jnp.where(sel0, col, P), axis=1, keepdims=True)
        sel = col == idx
        pxs.append(jnp.sum(jnp.where(sel, px, 0.0), axis=1, keepdims=True))
        pys.append(jnp.sum(jnp.where(sel, py, 0.0), axis=1, keepdims=True))
        pzs.append(jnp.sum(jnp.where(sel, pz, 0.0), axis=1, keepdims=True))
        ds.append(jnp.sqrt(m))
        d2 = jnp.where(sel, big, d2)

    d = jnp.concatenate(ds, axis=1)
    pxc = jnp.concatenate(pxs, axis=1)
    pyc = jnp.concatenate(pys, axis=1)
    pzc = jnp.concatenate(pzs, axis=1)

    rx = pxc - qx
    ry = pyc - qy
    rz = pzc - qz
    inv = 1.0 / jnp.maximum(d, 1e-12)
    rdx_ref[...] = rx * inv
    rdy_ref[...] = ry * inv
    rdz_ref[...] = rz * inv
    d_ref[...] = d
    c = rz / (d + EPS_A)
    el = _acos(c)
    sinel = jnp.sqrt(jnp.maximum(1.0 - c * c, 0.0))
    az = _acos(rx / (d * sinel + EPS_A))
    az = jnp.where(ry < 0.0, 2.0 * math.pi - az, az)
    az_ref[...] = az
    el_ref[...] = el


def kernel(ray_o, light_probe_pos, *, interpret=False):
    Q = ray_o.shape[0]
    P = light_probe_pos.shape[0]
    bQ = 128
    grid = (Q // bQ,)

    qcols = [ray_o[:, i].reshape(Q, 1) for i in range(3)]
    pcols = [light_probe_pos[:, i].reshape(1, P) for i in range(3)]

    qspec = pl.BlockSpec((bQ, 1), lambda i: (i, 0))
    pspec = pl.BlockSpec((1, P), lambda i: (0, 0))
    ospec = pl.BlockSpec((bQ, K_C), lambda i: (i, 0))
    oshape = jax.ShapeDtypeStruct((Q, K_C), jnp.float32)

    outs = pl.pallas_call(
        functools.partial(_knn_body, P=P),
        grid=grid,
        in_specs=[qspec] * 3 + [pspec] * 3,
        out_specs=[ospec] * 6,
        out_shape=[oshape] * 6,
        interpret=interpret,
    )(*qcols, *pcols)

    rdx, rdy, rdz, d, az, el = outs
    return jnp.stack([rdx, rdy, rdz, d, az, el], axis=-1)


# speedup vs baseline: 3.8775x; 1.4659x over previous
import functools
import math

import jax
import jax.numpy as jnp
from jax import lax
from jax.experimental import pallas as pl
from jax.experimental.pallas import tpu as pltpu
from jax.experimental.pallas import tpu_sc as plsc

K_C = 16
EPS_A = 1e-5
BIG = 3.0e38


def _acos(x):
    x = jnp.clip(x, -1.0, 1.0)
    ax = jnp.abs(x)
    p = jnp.float32(-0.0012624911)
    for c in (0.0066700901, -0.0170881256, 0.0308918810, -0.0501743046,
              0.0889789874, -0.2145988016, 1.5707963050):
        p = p * ax + jnp.float32(c)
    r = jnp.sqrt(jnp.maximum(1.0 - ax, 0.0)) * p
    return jnp.where(x < 0.0, jnp.float32(math.pi) - r, r)


def _pair_min(v, i, h):
    a, b = v[:h], v[h:]
    ia, ib = i[:h], i[h:]
    c = a <= b
    return jnp.where(c, a, b), jnp.where(c, ia, ib)


def _topk_body(qx_ref, qy_ref, qz_ref, px_ref, py_ref, pz_ref,
               d_ref, idx_ref, *, P):
    qx = qx_ref[...]
    qy = qy_ref[...]
    qz = qz_ref[...]
    px = px_ref[...]
    py = py_ref[...]
    pz = pz_ref[...]

    dx = px - qx
    dy = py - qy
    dz = pz - qz
    d2 = dx * dx + dy * dy + dz * dz
    riota = lax.broadcasted_iota(jnp.int32, d2.shape, 0)

    cur = d2
    ds, idxs = [], []
    for _ in range(K_C):
        v, i = cur, riota
        n = P
        while n > 1:
            h = n // 2
            v, i = _pair_min(v, i, h)
            n = h
        m, mi = v, i
        ds.append(jnp.sqrt(m))
        idxs.append(mi)
        cur = jnp.where(d2 > m, d2, BIG)

    d_ref[...] = jnp.concatenate(ds, axis=0)
    idx_ref[...] = jnp.concatenate(idxs, axis=0)


def _topk_call(qcols, pcols, Q, P, bQ):
    grid = (Q // bQ,)
    qspec = pl.BlockSpec((1, bQ), lambda i: (0, i))
    pspec = pl.BlockSpec((P, 1), lambda i: (0, 0))
    ospec = pl.BlockSpec((K_C, bQ), lambda i: (0, i))
    return pl.pallas_call(
        functools.partial(_topk_body, P=P),
        grid=grid,
        in_specs=[qspec] * 3 + [pspec] * 3,
        out_specs=[ospec, ospec],
        out_shape=[jax.ShapeDtypeStruct((K_C, Q), jnp.float32),
                   jax.ShapeDtypeStruct((K_C, Q), jnp.int32)],
    )(*qcols, *pcols)


def _gather_call(idx_flat, pcols_flat, P):
    N = idx_flat.shape[0]
    info = plsc.get_sparse_core_info()
    NC, NS = info.num_cores, info.num_subcores
    NW = NC * NS
    n_w = N // NW
    mesh = plsc.VectorSubcoreMesh(core_axis_name="c", subcore_axis_name="s")

    @functools.partial(
        pl.kernel, mesh=mesh,
        compiler_params=pltpu.CompilerParams(needs_layout_passes=False),
        out_type=[jax.ShapeDtypeStruct((N,), jnp.float32)] * 3,
        scratch_types=[pltpu.VMEM((n_w,), jnp.int32)]
                      + [pltpu.VMEM((P,), jnp.float32)] * 3
                      + [pltpu.VMEM((n_w,), jnp.float32)] * 3,
    )
    def gather_k(idx_hbm, px_hbm, py_hbm, pz_hbm,
                 ox_hbm, oy_hbm, oz_hbm,
                 idx_v, px_v, py_v, pz_v, ox_v, oy_v, oz_v):
        wid = lax.axis_index("s") * NC + lax.axis_index("c")
        base = wid * n_w
        pltpu.sync_copy(idx_hbm.at[pl.ds(base, n_w)], idx_v)
        pltpu.sync_copy(px_hbm, px_v)
        pltpu.sync_copy(py_hbm, py_v)
        pltpu.sync_copy(pz_hbm, pz_v)

        def body(j, carry):
            o = j * 16
            iv = idx_v[pl.ds(o, 16)]
            ox_v[pl.ds(o, 16)] = plsc.load_gather(px_v, [iv])
            oy_v[pl.ds(o, 16)] = plsc.load_gather(py_v, [iv])
            oz_v[pl.ds(o, 16)] = plsc.load_gather(pz_v, [iv])
            return carry

        lax.fori_loop(0, n_w // 16, body, 0)
        pltpu.sync_copy(ox_v, ox_hbm.at[pl.ds(base, n_w)])
        pltpu.sync_copy(oy_v, oy_hbm.at[pl.ds(base, n_w)])
        pltpu.sync_copy(oz_v, oz_hbm.at[pl.ds(base, n_w)])

    return gather_k(idx_flat, *pcols_flat)


def _angles_body(qx_ref, qy_ref, qz_ref, gx_ref, gy_ref, gz_ref, d_ref,
                 rdx_ref, rdy_ref, rdz_ref, az_ref, el_ref):
    rx = gx_ref[...] - qx_ref[...]
    ry = gy_ref[...] - qy_ref[...]
    rz = gz_ref[...] - qz_ref[...]
    d = d_ref[...]
    inv = 1.0 / jnp.maximum(d, 1e-12)
    rdx_ref[...] = rx * inv
    rdy_ref[...] = ry * inv
    rdz_ref[...] = rz * inv
    c = rz / (d + EPS_A)
    el = _acos(c)
    sinel = jnp.sqrt(jnp.maximum(1.0 - c * c, 0.0))
    az = _acos(rx / (d * sinel + EPS_A))
    az_ref[...] = jnp.where(ry < 0.0, 2.0 * math.pi - az, az)
    el_ref[...] = el


def _angles_call(qcols, gx, gy, gz, d, Q, bq):
    grid = (Q // bq,)
    qspec = pl.BlockSpec((1, bq), lambda i: (0, i))
    kspec = pl.BlockSpec((K_C, bq), lambda i: (0, i))
    return pl.pallas_call(
        _angles_body,
        grid=grid,
        in_specs=[qspec] * 3 + [kspec] * 4,
        out_specs=[kspec] * 5,
        out_shape=[jax.ShapeDtypeStruct((K_C, Q), jnp.float32)] * 5,
    )(*qcols, gx, gy, gz, d)


def kernel(ray_o, light_probe_pos):
    Q = ray_o.shape[0]
    P = light_probe_pos.shape[0]

    qcols = [ray_o[:, i].reshape(1, Q) for i in range(3)]
    pcols = [light_probe_pos[:, i].reshape(P, 1) for i in range(3)]
    pcols_flat = [light_probe_pos[:, i].reshape(P) for i in range(3)]

    d, idx = _topk_call(qcols, pcols, Q, P, bQ=128)

    gx, gy, gz = _gather_call(idx.reshape(-1), pcols_flat, P)
    gx = gx.reshape(K_C, Q)
    gy = gy.reshape(K_C, Q)
    gz = gz.reshape(K_C, Q)

    rdx, rdy, rdz, az, el = _angles_call(qcols, gx, gy, gz, d, Q, bq=2048)

    out = jnp.stack([rdx, rdy, rdz, d, az, el], axis=-1)
    return out.transpose(1, 0, 2)


# speedup vs baseline: 4.6575x; 1.2012x over previous
import functools
import math

import jax
import jax.numpy as jnp
from jax import lax
from jax.experimental import pallas as pl
from jax.experimental.pallas import tpu as pltpu
from jax.experimental.pallas import tpu_sc as plsc

K_C = 16
EPS_A = 1e-5
BIG = 3.0e38


def _acos(x):
    x = jnp.clip(x, -1.0, 1.0)
    ax = jnp.abs(x)
    p = jnp.float32(-0.0012624911)
    for c in (0.0066700901, -0.0170881256, 0.0308918810, -0.0501743046,
              0.0889789874, -0.2145988016, 1.5707963050):
        p = p * ax + jnp.float32(c)
    r = jnp.sqrt(jnp.maximum(1.0 - ax, 0.0)) * p
    return jnp.where(x < 0.0, jnp.float32(math.pi) - r, r)


def _pair_min(v, i, h):
    a, b = v[:h], v[h:]
    ia, ib = i[:h], i[h:]
    c = a <= b
    return jnp.where(c, a, b), jnp.where(c, ia, ib)


def _topk_body(qx_ref, qy_ref, qz_ref, px_ref, py_ref, pz_ref,
               d_ref, idx_ref, *, P, CHUNK=64):
    qx = qx_ref[...]
    qy = qy_ref[...]
    qz = qz_ref[...]
    px = px_ref[...]
    py = py_ref[...]
    pz = pz_ref[...]

    dx = px - qx
    dy = py - qy
    dz = pz - qz
    d2 = dx * dx + dy * dy + dz * dz
    riota = lax.broadcasted_iota(jnp.int32, d2.shape, 0)

    m_prev = jnp.float32(-1.0)
    ds, idxs = [], []
    for _ in range(K_C):
        acc_v = acc_i = None
        for c in range(0, P, CHUNK):
            v = d2[c:c + CHUNK]
            i = riota[c:c + CHUNK]
            v = jnp.where(v > m_prev, v, BIG)
            n = CHUNK
            while n > 8:
                h = n // 2
                v, i = _pair_min(v, i, h)
                n = h
            if acc_v is None:
                acc_v, acc_i = v, i
            else:
                cnd = acc_v <= v
                acc_v = jnp.where(cnd, acc_v, v)
                acc_i = jnp.where(cnd, acc_i, i)
        v, i = acc_v, acc_i
        n = 8
        while n > 1:
            h = n // 2
            v, i = _pair_min(v, i, h)
            n = h
        m, mi = v, i
        ds.append(jnp.sqrt(m))
        idxs.append(mi)
        m_prev = m

    d_ref[...] = jnp.concatenate(ds, axis=0)
    idx_ref[...] = jnp.concatenate(idxs, axis=0)


def _topk_call(qcols, pcols, Q, P, bQ):
    grid = (Q // bQ,)
    qspec = pl.BlockSpec((1, bQ), lambda i: (0, i))
    pspec = pl.BlockSpec((P, 1), lambda i: (0, 0))
    ospec = pl.BlockSpec((K_C, bQ), lambda i: (0, i))
    return pl.pallas_call(
        functools.partial(_topk_body, P=P),
        grid=grid,
        in_specs=[qspec] * 3 + [pspec] * 3,
        out_specs=[ospec, ospec],
        out_shape=[jax.ShapeDtypeStruct((K_C, Q), jnp.float32),
                   jax.ShapeDtypeStruct((K_C, Q), jnp.int32)],
    )(*qcols, *pcols)


def _gather_call(idx_flat, pcols_flat, P):
    N = idx_flat.shape[0]
    info = plsc.get_sparse_core_info()
    NC, NS = info.num_cores, info.num_subcores
    NW = NC * NS
    n_w = N // NW
    mesh = plsc.VectorSubcoreMesh(core_axis_name="c", subcore_axis_name="s")

    @functools.partial(
        pl.kernel, mesh=mesh,
        compiler_params=pltpu.CompilerParams(needs_layout_passes=False),
        out_type=[jax.ShapeDtypeStruct((N,), jnp.float32)] * 3,
        scratch_types=[pltpu.VMEM((n_w,), jnp.int32)]
                      + [pltpu.VMEM((P,), jnp.float32)] * 3
                      + [pltpu.VMEM((n_w,), jnp.float32)] * 3,
    )
    def gather_k(idx_hbm, px_hbm, py_hbm, pz_hbm,
                 ox_hbm, oy_hbm, oz_hbm,
                 idx_v, px_v, py_v, pz_v, ox_v, oy_v, oz_v):
        wid = lax.axis_index("s") * NC + lax.axis_index("c")
        base = wid * n_w
        pltpu.sync_copy(idx_hbm.at[pl.ds(base, n_w)], idx_v)
        pltpu.sync_copy(px_hbm, px_v)
        pltpu.sync_copy(py_hbm, py_v)
        pltpu.sync_copy(pz_hbm, pz_v)

        def body(j, carry):
            o = j * 16
            iv = idx_v[pl.ds(o, 16)]
            ox_v[pl.ds(o, 16)] = plsc.load_gather(px_v, [iv])
            oy_v[pl.ds(o, 16)] = plsc.load_gather(py_v, [iv])
            oz_v[pl.ds(o, 16)] = plsc.load_gather(pz_v, [iv])
            return carry

        lax.fori_loop(0, n_w // 16, body, 0)
        pltpu.sync_copy(ox_v, ox_hbm.at[pl.ds(base, n_w)])
        pltpu.sync_copy(oy_v, oy_hbm.at[pl.ds(base, n_w)])
        pltpu.sync_copy(oz_v, oz_hbm.at[pl.ds(base, n_w)])

    return gather_k(idx_flat, *pcols_flat)


def _angles_body(qx_ref, qy_ref, qz_ref, gx_ref, gy_ref, gz_ref, d_ref,
                 rdx_ref, rdy_ref, rdz_ref, az_ref, el_ref):
    rx = gx_ref[...] - qx_ref[...]
    ry = gy_ref[...] - qy_ref[...]
    rz = gz_ref[...] - qz_ref[...]
    d = d_ref[...]
    inv = 1.0 / jnp.maximum(d, 1e-12)
    rdx_ref[...] = rx * inv
    rdy_ref[...] = ry * inv
    rdz_ref[...] = rz * inv
    c = rz / (d + EPS_A)
    el = _acos(c)
    sinel = jnp.sqrt(jnp.maximum(1.0 - c * c, 0.0))
    az = _acos(rx / (d * sinel + EPS_A))
    az_ref[...] = jnp.where(ry < 0.0, 2.0 * math.pi - az, az)
    el_ref[...] = el


def _angles_call(qcols, gx, gy, gz, d, Q, bq):
    grid = (Q // bq,)
    qspec = pl.BlockSpec((1, bq), lambda i: (0, i))
    kspec = pl.BlockSpec((K_C, bq), lambda i: (0, i))
    return pl.pallas_call(
        _angles_body,
        grid=grid,
        in_specs=[qspec] * 3 + [kspec] * 4,
        out_specs=[kspec] * 5,
        out_shape=[jax.ShapeDtypeStruct((K_C, Q), jnp.float32)] * 5,
    )(*qcols, gx, gy, gz, d)


def kernel(ray_o, light_probe_pos):
    Q = ray_o.shape[0]
    P = light_probe_pos.shape[0]

    qcols = [ray_o[:, i].reshape(1, Q) for i in range(3)]
    pcols = [light_probe_pos[:, i].reshape(P, 1) for i in range(3)]
    pcols_flat = [light_probe_pos[:, i].reshape(P) for i in range(3)]

    d, idx = _topk_call(qcols, pcols, Q, P, bQ=128)

    gx, gy, gz = _gather_call(idx.reshape(-1), pcols_flat, P)
    gx = gx.reshape(K_C, Q)
    gy = gy.reshape(K_C, Q)
    gz = gz.reshape(K_C, Q)

    rdx, rdy, rdz, az, el = _angles_call(qcols, gx, gy, gz, d, Q, bq=2048)

    out = jnp.stack([rdx, rdy, rdz, d, az, el], axis=-1)
    return out.transpose(1, 0, 2)


# speedup vs baseline: 6.4466x; 1.3841x over previous
import functools
import math

import jax
import jax.numpy as jnp
from jax import lax
from jax.experimental import pallas as pl
from jax.experimental.pallas import tpu as pltpu
from jax.experimental.pallas import tpu_sc as plsc

K_C = 16
EPS_A = 1e-5
BIG = 3.0e38


def _acos(x):
    x = jnp.clip(x, -1.0, 1.0)
    ax = jnp.abs(x)
    p = jnp.float32(-0.0012624911)
    for c in (0.0066700901, -0.0170881256, 0.0308918810, -0.0501743046,
              0.0889789874, -0.2145988016, 1.5707963050):
        p = p * ax + jnp.float32(c)
    r = jnp.sqrt(jnp.maximum(1.0 - ax, 0.0)) * p
    return jnp.where(x < 0.0, jnp.float32(math.pi) - r, r)


IMAX = 0x7FFFFFFF


def _topk_body(qx_ref, qy_ref, qz_ref, px_ref, py_ref, pz_ref,
               idx_ref, *, P, CHUNK=64):
    qx = qx_ref[...]
    qy = qy_ref[...]
    qz = qz_ref[...]
    px = px_ref[...]
    py = py_ref[...]
    pz = pz_ref[...]

    dx = px - qx
    dy = py - qy
    dz = pz - qz
    d2 = dx * dx + dy * dy + dz * dz
    riota = lax.broadcasted_iota(jnp.int32, d2.shape, 0)
    key = (lax.bitcast_convert_type(d2, jnp.int32) & jnp.int32(-P)) | riota

    mp = jnp.int32(-1)
    idxs = []
    for _ in range(K_C):
        acc = None
        for c in range(0, P, CHUNK):
            v = key[c:c + CHUNK]
            v = jnp.where(v > mp, v, IMAX)
            n = CHUNK
            while n > 8:
                h = n // 2
                v = jnp.minimum(v[:h], v[h:])
                n = h
            acc = v if acc is None else jnp.minimum(acc, v)
        n = 8
        while n > 1:
            h = n // 2
            acc = jnp.minimum(acc[:h], acc[h:])
            n = h
        mp = acc
        idxs.append(mp & jnp.int32(P - 1))

    idx_ref[...] = jnp.concatenate(idxs, axis=0)


def _topk_call(qcols, pcols, Q, P, bQ):
    grid = (Q // bQ,)
    qspec = pl.BlockSpec((1, bQ), lambda i: (0, i))
    pspec = pl.BlockSpec((P, 1), lambda i: (0, 0))
    ospec = pl.BlockSpec((K_C, bQ), lambda i: (0, i))
    return pl.pallas_call(
        functools.partial(_topk_body, P=P),
        grid=grid,
        in_specs=[qspec] * 3 + [pspec] * 3,
        out_specs=[ospec],
        out_shape=[jax.ShapeDtypeStruct((K_C, Q), jnp.int32)],
    )(*qcols, *pcols)[0]


def _gather_call(idx_flat, pcols_flat, qreps, P):
    N = idx_flat.shape[0]
    info = plsc.get_sparse_core_info()
    NC, NS = info.num_cores, info.num_subcores
    NW = NC * NS
    n_w = N // NW
    mesh = plsc.VectorSubcoreMesh(core_axis_name="c", subcore_axis_name="s")

    @functools.partial(
        pl.kernel, mesh=mesh,
        compiler_params=pltpu.CompilerParams(needs_layout_passes=False),
        out_type=[jax.ShapeDtypeStruct((N,), jnp.float32)] * 3,
        scratch_types=[pltpu.VMEM((n_w,), jnp.int32)]
                      + [pltpu.VMEM((P,), jnp.float32)] * 3
                      + [pltpu.VMEM((n_w,), jnp.float32)] * 3
                      + [pltpu.VMEM((n_w,), jnp.float32)] * 3,
    )
    def gather_k(idx_hbm, px_hbm, py_hbm, pz_hbm, qx_hbm, qy_hbm, qz_hbm,
                 ox_hbm, oy_hbm, oz_hbm,
                 idx_v, px_v, py_v, pz_v,
                 qx_v, qy_v, qz_v, ox_v, oy_v, oz_v):
        wid = lax.axis_index("s") * NC + lax.axis_index("c")
        base = wid * n_w
        pltpu.sync_copy(idx_hbm.at[pl.ds(base, n_w)], idx_v)
        pltpu.sync_copy(px_hbm, px_v)
        pltpu.sync_copy(py_hbm, py_v)
        pltpu.sync_copy(pz_hbm, pz_v)
        pltpu.sync_copy(qx_hbm.at[pl.ds(base, n_w)], qx_v)
        pltpu.sync_copy(qy_hbm.at[pl.ds(base, n_w)], qy_v)
        pltpu.sync_copy(qz_hbm.at[pl.ds(base, n_w)], qz_v)

        def body(j, carry):
            o = j * 16
            sl = pl.ds(o, 16)
            iv = idx_v[sl]
            gx = plsc.load_gather(px_v, [iv])
            gy = plsc.load_gather(py_v, [iv])
            gz = plsc.load_gather(pz_v, [iv])
            rx = gx - qx_v[sl]
            ry = gy - qy_v[sl]
            rz = gz - qz_v[sl]
            e2 = rx * rx + ry * ry + rz * rz
            _, sx = plsc.sort_key_val(e2, gx)
            _, sy = plsc.sort_key_val(e2, gy)
            _, sz = plsc.sort_key_val(e2, gz)
            ox_v[sl] = sx
            oy_v[sl] = sy
            oz_v[sl] = sz
            return carry

        lax.fori_loop(0, n_w // 16, body, 0)
        pltpu.sync_copy(ox_v, ox_hbm.at[pl.ds(base, n_w)])
        pltpu.sync_copy(oy_v, oy_hbm.at[pl.ds(base, n_w)])
        pltpu.sync_copy(oz_v, oz_hbm.at[pl.ds(base, n_w)])

    return gather_k(idx_flat, *pcols_flat, *qreps)


def _angles_body(qx_ref, qy_ref, qz_ref, gx_ref, gy_ref, gz_ref,
                 rdx_ref, rdy_ref, rdz_ref, d_ref, az_ref, el_ref):
    rx = gx_ref[...] - qx_ref[...]
    ry = gy_ref[...] - qy_ref[...]
    rz = gz_ref[...] - qz_ref[...]
    d = jnp.sqrt(rx * rx + ry * ry + rz * rz)
    d_ref[...] = d
    inv = 1.0 / jnp.maximum(d, 1e-12)
    rdx_ref[...] = rx * inv
    rdy_ref[...] = ry * inv
    rdz_ref[...] = rz * inv
    c = rz / (d + EPS_A)
    el = _acos(c)
    sinel = jnp.sqrt(jnp.maximum(1.0 - c * c, 0.0))
    az = _acos(rx / (d * sinel + EPS_A))
    az_ref[...] = jnp.where(ry < 0.0, 2.0 * math.pi - az, az)
    el_ref[...] = el


def _angles_call(qcols, gx, gy, gz, Q, bq):
    grid = (Q // bq,)
    qspec = pl.BlockSpec((1, bq), lambda i: (0, i))
    kspec = pl.BlockSpec((K_C, bq), lambda i: (0, i))
    return pl.pallas_call(
        _angles_body,
        grid=grid,
        in_specs=[qspec] * 3 + [kspec] * 3,
        out_specs=[kspec] * 6,
        out_shape=[jax.ShapeDtypeStruct((K_C, Q), jnp.float32)] * 6,
    )(*qcols, gx, gy, gz)


def kernel(ray_o, light_probe_pos):
    Q = ray_o.shape[0]
    P = light_probe_pos.shape[0]

    qcols = [ray_o[:, i].reshape(1, Q) for i in range(3)]
    pcols = [light_probe_pos[:, i].reshape(P, 1) for i in range(3)]
    pcols_flat = [light_probe_pos[:, i].reshape(P) for i in range(3)]

    idx = _topk_call(qcols, pcols, Q, P, bQ=128)

    qreps = [jnp.repeat(ray_o[:, i], K_C) for i in range(3)]
    gx, gy, gz = _gather_call(idx.T.reshape(-1), pcols_flat, qreps, P)
    gx = gx.reshape(Q, K_C).T
    gy = gy.reshape(Q, K_C).T
    gz = gz.reshape(Q, K_C).T

    rdx, rdy, rdz, d, az, el = _angles_call(qcols, gx, gy, gz, Q, bq=2048)

    out = jnp.stack([rdx, rdy, rdz, d, az, el], axis=-1)
    return out.transpose(1, 0, 2)


# speedup vs baseline: 6.7785x; 1.0515x over previous
import functools
import math

import jax
import jax.numpy as jnp
from jax import lax
from jax.experimental import pallas as pl
from jax.experimental.pallas import tpu as pltpu
from jax.experimental.pallas import tpu_sc as plsc

K_C = 16
EPS_A = 1e-5
BIG = 3.0e38


def _acos(x):
    x = jnp.clip(x, -1.0, 1.0)
    ax = jnp.abs(x)
    p = jnp.float32(-0.0012624911)
    for c in (0.0066700901, -0.0170881256, 0.0308918810, -0.0501743046,
              0.0889789874, -0.2145988016, 1.5707963050):
        p = p * ax + jnp.float32(c)
    r = jnp.sqrt(jnp.maximum(1.0 - ax, 0.0)) * p
    return jnp.where(x < 0.0, jnp.float32(math.pi) - r, r)


IMAX = 0x7FFFFFFF


def _topk_body(qx_ref, qy_ref, qz_ref, px_ref, py_ref, pz_ref,
               idx_ref, *, P, CHUNK=64):
    qx = qx_ref[...]
    qy = qy_ref[...]
    qz = qz_ref[...]
    px = px_ref[...]
    py = py_ref[...]
    pz = pz_ref[...]

    dx = px - qx
    dy = py - qy
    dz = pz - qz
    d2 = dx * dx + dy * dy + dz * dz
    riota = lax.broadcasted_iota(jnp.int32, d2.shape, 0)
    keyi = (lax.bitcast_convert_type(d2, jnp.int32) & jnp.int32(-P)) | riota
    key = lax.bitcast_convert_type(keyi, jnp.float32)
    pad = float(jnp.finfo(jnp.float32).max)

    mp = jnp.float32(-1.0)
    idxs = []
    for _ in range(K_C):
        acc = None
        for c in range(0, P, CHUNK):
            v = key[c:c + CHUNK]
            v = jnp.where(v > mp, v, pad)
            n = CHUNK
            while n > 8:
                h = n // 2
                v = jnp.minimum(v[:h], v[h:])
                n = h
            acc = v if acc is None else jnp.minimum(acc, v)
        n = 8
        while n > 1:
            h = n // 2
            acc = jnp.minimum(acc[:h], acc[h:])
            n = h
        mp = acc
        idxs.append(lax.bitcast_convert_type(mp, jnp.int32)
                    & jnp.int32(P - 1))

    idx_ref[...] = jnp.concatenate(idxs, axis=0)


def _topk_call(qcols, pcols, Q, P, bQ):
    grid = (Q // bQ,)
    qspec = pl.BlockSpec((1, bQ), lambda i: (0, i))
    pspec = pl.BlockSpec((P, 1), lambda i: (0, 0))
    ospec = pl.BlockSpec((K_C, bQ), lambda i: (0, i))
    return pl.pallas_call(
        functools.partial(_topk_body, P=P),
        grid=grid,
        in_specs=[qspec] * 3 + [pspec] * 3,
        out_specs=[ospec],
        out_shape=[jax.ShapeDtypeStruct((K_C, Q), jnp.int32)],
    )(*qcols, *pcols)[0]


def _gather_call(idx_flat, pcols_flat, qreps, P):
    N = idx_flat.shape[0]
    info = plsc.get_sparse_core_info()
    NC, NS = info.num_cores, info.num_subcores
    NW = NC * NS
    n_w = N // NW
    mesh = plsc.VectorSubcoreMesh(core_axis_name="c", subcore_axis_name="s")

    @functools.partial(
        pl.kernel, mesh=mesh,
        compiler_params=pltpu.CompilerParams(needs_layout_passes=False),
        out_type=[jax.ShapeDtypeStruct((N,), jnp.float32)] * 3,
        scratch_types=[pltpu.VMEM((n_w,), jnp.int32)]
                      + [pltpu.VMEM((P,), jnp.float32)] * 3
                      + [pltpu.VMEM((n_w,), jnp.float32)] * 3
                      + [pltpu.VMEM((n_w,), jnp.float32)] * 3,
    )
    def gather_k(idx_hbm, px_hbm, py_hbm, pz_hbm, qx_hbm, qy_hbm, qz_hbm,
                 ox_hbm, oy_hbm, oz_hbm,
                 idx_v, px_v, py_v, pz_v,
                 qx_v, qy_v, qz_v, ox_v, oy_v, oz_v):
        wid = lax.axis_index("s") * NC + lax.axis_index("c")
        base = wid * n_w
        pltpu.sync_copy(idx_hbm.at[pl.ds(base, n_w)], idx_v)
        pltpu.sync_copy(px_hbm, px_v)
        pltpu.sync_copy(py_hbm, py_v)
        pltpu.sync_copy(pz_hbm, pz_v)
        pltpu.sync_copy(qx_hbm.at[pl.ds(base, n_w)], qx_v)
        pltpu.sync_copy(qy_hbm.at[pl.ds(base, n_w)], qy_v)
        pltpu.sync_copy(qz_hbm.at[pl.ds(base, n_w)], qz_v)

        def body(j, carry):
            o = j * 16
            sl = pl.ds(o, 16)
            iv = idx_v[sl]
            gx = plsc.load_gather(px_v, [iv])
            gy = plsc.load_gather(py_v, [iv])
            gz = plsc.load_gather(pz_v, [iv])
            rx = gx - qx_v[sl]
            ry = gy - qy_v[sl]
            rz = gz - qz_v[sl]
            e2 = rx * rx + ry * ry + rz * rz
            _, sx = plsc.sort_key_val(e2, rx)
            _, sy = plsc.sort_key_val(e2, ry)
            _, sz = plsc.sort_key_val(e2, rz)
            ox_v[sl] = sx
            oy_v[sl] = sy
            oz_v[sl] = sz
            return carry

        lax.fori_loop(0, n_w // 16, body, 0)
        pltpu.sync_copy(ox_v, ox_hbm.at[pl.ds(base, n_w)])
        pltpu.sync_copy(oy_v, oy_hbm.at[pl.ds(base, n_w)])
        pltpu.sync_copy(oz_v, oz_hbm.at[pl.ds(base, n_w)])

    return gather_k(idx_flat, *pcols_flat, *qreps)


def _angles_body(rx_ref, ry_ref, rz_ref,
                 rdx_ref, rdy_ref, rdz_ref, d_ref, az_ref, el_ref):
    rx = rx_ref[...]
    ry = ry_ref[...]
    rz = rz_ref[...]
    d = jnp.sqrt(rx * rx + ry * ry + rz * rz)
    d_ref[...] = d
    inv = 1.0 / jnp.maximum(d, 1e-12)
    rdx_ref[...] = rx * inv
    rdy_ref[...] = ry * inv
    rdz_ref[...] = rz * inv
    c = rz / (d + EPS_A)
    el = _acos(c)
    sinel = jnp.sqrt(jnp.maximum(1.0 - c * c, 0.0))
    az = _acos(rx / (d * sinel + EPS_A))
    az_ref[...] = jnp.where(ry < 0.0, 2.0 * math.pi - az, az)
    el_ref[...] = el


def _angles_call(rx, ry, rz, rows, cols, brows):
    grid = (rows // brows,)
    spec = pl.BlockSpec((brows, cols), lambda i: (i, 0))
    return pl.pallas_call(
        _angles_body,
        grid=grid,
        in_specs=[spec] * 3,
        out_specs=[spec] * 6,
        out_shape=[jax.ShapeDtypeStruct((rows, cols), jnp.float32)] * 6,
    )(rx, ry, rz)


def kernel(ray_o, light_probe_pos):
    Q = ray_o.shape[0]
    P = light_probe_pos.shape[0]

    qcols = [ray_o[:, i].reshape(1, Q) for i in range(3)]
    pcols = [light_probe_pos[:, i].reshape(P, 1) for i in range(3)]
    pcols_flat = [light_probe_pos[:, i].reshape(P) for i in range(3)]

    idx = _topk_call(qcols, pcols, Q, P, bQ=128)

    qreps = [jnp.repeat(ray_o[:, i], K_C) for i in range(3)]
    srx, sry, srz = _gather_call(idx.T.reshape(-1), pcols_flat, qreps, P)

    N = Q * K_C
    cols = 2048
    rows = N // cols
    outs = _angles_call(srx.reshape(rows, cols), sry.reshape(rows, cols),
                        srz.reshape(rows, cols), rows, cols, brows=rows // 8)

    chans = [o.reshape(Q, K_C) for o in outs]
    return jnp.stack(chans, axis=-1)


# speedup vs baseline: 7.5431x; 1.1128x over previous
import functools
import math

import jax
import jax.numpy as jnp
from jax import lax
from jax.experimental import pallas as pl
from jax.experimental.pallas import tpu as pltpu
from jax.experimental.pallas import tpu_sc as plsc

K_C = 16
EPS_A = 1e-5
BIG = 3.0e38


def _acos(x):
    x = jnp.clip(x, -1.0, 1.0)
    ax = jnp.abs(x)
    p = jnp.float32(-0.0012624911)
    for c in (0.0066700901, -0.0170881256, 0.0308918810, -0.0501743046,
              0.0889789874, -0.2145988016, 1.5707963050):
        p = p * ax + jnp.float32(c)
    r = jnp.sqrt(jnp.maximum(1.0 - ax, 0.0)) * p
    return jnp.where(x < 0.0, jnp.float32(math.pi) - r, r)


IMAX = 0x7FFFFFFF


def _topk_body(qx_ref, qy_ref, qz_ref, px_ref, py_ref, pz_ref,
               idx_ref, *, P, CHUNK=64):
    qx = qx_ref[...]
    qy = qy_ref[...]
    qz = qz_ref[...]
    px = px_ref[...]
    py = py_ref[...]
    pz = pz_ref[...]

    dx = px - qx
    dy = py - qy
    dz = pz - qz
    d2 = dx * dx + dy * dy + dz * dz
    riota = lax.broadcasted_iota(jnp.int32, d2.shape, 0)
    keyi = (lax.bitcast_convert_type(d2, jnp.int32) & jnp.int32(-P)) | riota
    key = lax.bitcast_convert_type(keyi, jnp.float32)
    pad = float(jnp.finfo(jnp.float32).max)

    mp = jnp.float32(-1.0)
    idxs = []
    for _ in range(K_C):
        acc = None
        for c in range(0, P, CHUNK):
            v = key[c:c + CHUNK]
            v = jnp.where(v > mp, v, pad)
            n = CHUNK
            while n > 8:
                h = n // 2
                v = jnp.minimum(v[:h], v[h:])
                n = h
            acc = v if acc is None else jnp.minimum(acc, v)
        n = 8
        while n > 1:
            h = n // 2
            acc = jnp.minimum(acc[:h], acc[h:])
            n = h
        mp = acc
        idxs.append(lax.bitcast_convert_type(mp, jnp.int32)
                    & jnp.int32(P - 1))

    idx_ref[...] = jnp.concatenate(idxs, axis=0)


def _topk_call(qcols, pcols, Q, P, bQ):
    grid = (Q // bQ,)
    qspec = pl.BlockSpec((1, bQ), lambda i: (0, i))
    pspec = pl.BlockSpec((P, 1), lambda i: (0, 0))
    ospec = pl.BlockSpec((K_C, bQ), lambda i: (0, i))
    return pl.pallas_call(
        functools.partial(_topk_body, P=P),
        grid=grid,
        in_specs=[qspec] * 3 + [pspec] * 3,
        out_specs=[ospec],
        out_shape=[jax.ShapeDtypeStruct((K_C, Q), jnp.int32)],
    )(*qcols, *pcols)[0]


def _gather_call(idx_flat, pcols_flat, qreps, P):
    N = idx_flat.shape[0]
    info = plsc.get_sparse_core_info()
    NC, NS = info.num_cores, info.num_subcores
    NW = NC * NS
    n_w = N // NW
    mesh = plsc.VectorSubcoreMesh(core_axis_name="c", subcore_axis_name="s")

    @functools.partial(
        pl.kernel, mesh=mesh,
        compiler_params=pltpu.CompilerParams(needs_layout_passes=False),
        out_type=[jax.ShapeDtypeStruct((N,), jnp.float32)] * 3,
        scratch_types=[pltpu.VMEM((n_w,), jnp.int32)]
                      + [pltpu.VMEM((P,), jnp.float32)] * 3
                      + [pltpu.VMEM((n_w,), jnp.float32)] * 3
                      + [pltpu.VMEM((n_w,), jnp.float32)] * 3,
    )
    def gather_k(idx_hbm, px_hbm, py_hbm, pz_hbm, qx_hbm, qy_hbm, qz_hbm,
                 ox_hbm, oy_hbm, oz_hbm,
                 idx_v, px_v, py_v, pz_v,
                 qx_v, qy_v, qz_v, ox_v, oy_v, oz_v):
        wid = lax.axis_index("s") * NC + lax.axis_index("c")
        base = wid * n_w
        pltpu.sync_copy(idx_hbm.at[pl.ds(base, n_w)], idx_v)
        pltpu.sync_copy(px_hbm, px_v)
        pltpu.sync_copy(py_hbm, py_v)
        pltpu.sync_copy(pz_hbm, pz_v)
        pltpu.sync_copy(qx_hbm.at[pl.ds(base, n_w)], qx_v)
        pltpu.sync_copy(qy_hbm.at[pl.ds(base, n_w)], qy_v)
        pltpu.sync_copy(qz_hbm.at[pl.ds(base, n_w)], qz_v)

        def body(j, carry):
            o = j * 16
            sl = pl.ds(o, 16)
            iv = idx_v[sl]
            gx = plsc.load_gather(px_v, [iv])
            gy = plsc.load_gather(py_v, [iv])
            gz = plsc.load_gather(pz_v, [iv])
            rx = gx - qx_v[sl]
            ry = gy - qy_v[sl]
            rz = gz - qz_v[sl]
            e2 = rx * rx + ry * ry + rz * rz
            _, sx = plsc.sort_key_val(e2, rx)
            _, sy = plsc.sort_key_val(e2, ry)
            _, sz = plsc.sort_key_val(e2, rz)
            ox_v[sl] = sx
            oy_v[sl] = sy
            oz_v[sl] = sz
            return carry

        lax.fori_loop(0, n_w // 16, body, 0)
        pltpu.sync_copy(ox_v, ox_hbm.at[pl.ds(base, n_w)])
        pltpu.sync_copy(oy_v, oy_hbm.at[pl.ds(base, n_w)])
        pltpu.sync_copy(oz_v, oz_hbm.at[pl.ds(base, n_w)])

    return gather_k(idx_flat, *pcols_flat, *qreps)


def _angles_body(rx_ref, ry_ref, rz_ref,
                 rdx_ref, rdy_ref, rdz_ref, d_ref, az_ref, el_ref):
    rx = rx_ref[...]
    ry = ry_ref[...]
    rz = rz_ref[...]
    d = jnp.sqrt(rx * rx + ry * ry + rz * rz)
    d_ref[...] = d
    inv = 1.0 / jnp.maximum(d, 1e-12)
    rdx_ref[...] = rx * inv
    rdy_ref[...] = ry * inv
    rdz_ref[...] = rz * inv
    c = rz / (d + EPS_A)
    el = _acos(c)
    sinel = jnp.sqrt(jnp.maximum(1.0 - c * c, 0.0))
    az = _acos(rx / (d * sinel + EPS_A))
    az_ref[...] = jnp.where(ry < 0.0, 2.0 * math.pi - az, az)
    el_ref[...] = el


def _angles_call(rx, ry, rz, rows, cols, brows):
    grid = (rows // brows,)
    spec = pl.BlockSpec((brows, cols), lambda i: (i, 0))
    return pl.pallas_call(
        _angles_body,
        grid=grid,
        in_specs=[spec] * 3,
        out_specs=[spec] * 6,
        out_shape=[jax.ShapeDtypeStruct((rows, cols), jnp.float32)] * 6,
    )(rx, ry, rz)


def kernel(ray_o, light_probe_pos):
    Q = ray_o.shape[0]
    P = light_probe_pos.shape[0]

    qcols = [ray_o[:, i].reshape(1, Q) for i in range(3)]
    pcols = [light_probe_pos[:, i].reshape(P, 1) for i in range(3)]
    pcols_flat = [light_probe_pos[:, i].reshape(P) for i in range(3)]

    idx = _topk_call(qcols, pcols, Q, P, bQ=256)

    qreps = [jnp.repeat(ray_o[:, i], K_C) for i in range(3)]
    srx, sry, srz = _gather_call(idx.T.reshape(-1), pcols_flat, qreps, P)

    N = Q * K_C
    cols = 2048
    rows = N // cols
    outs = _angles_call(srx.reshape(rows, cols), sry.reshape(rows, cols),
                        srz.reshape(rows, cols), rows, cols, brows=rows // 8)

    chans = [o.reshape(Q, K_C) for o in outs]
    return jnp.stack(chans, axis=-1)


# speedup vs baseline: 7.8780x; 1.0444x over previous
import functools
import math

import jax
import jax.numpy as jnp
from jax import lax
from jax.experimental import pallas as pl
from jax.experimental.pallas import tpu as pltpu
from jax.experimental.pallas import tpu_sc as plsc

K_C = 16
EPS_A = 1e-5
BIG = 3.0e38


def _acos(x):
    x = jnp.clip(x, -1.0, 1.0)
    ax = jnp.abs(x)
    p = jnp.float32(-0.0012624911)
    for c in (0.0066700901, -0.0170881256, 0.0308918810, -0.0501743046,
              0.0889789874, -0.2145988016, 1.5707963050):
        p = p * ax + jnp.float32(c)
    r = jnp.sqrt(jnp.maximum(1.0 - ax, 0.0)) * p
    return jnp.where(x < 0.0, jnp.float32(math.pi) - r, r)


IMAX = 0x7FFFFFFF


def _topk_body(qx_ref, qy_ref, qz_ref, px_ref, py_ref, pz_ref,
               idx_ref, *, P, CHUNK=64):
    qx = qx_ref[...]
    qy = qy_ref[...]
    qz = qz_ref[...]
    px = px_ref[...]
    py = py_ref[...]
    pz = pz_ref[...]

    dx = px - qx
    dy = py - qy
    dz = pz - qz
    d2 = dx * dx + dy * dy + dz * dz
    riota = lax.broadcasted_iota(jnp.int32, d2.shape, 0)
    keyi = (lax.bitcast_convert_type(d2, jnp.int32) & jnp.int32(-P)) | riota
    key = lax.bitcast_convert_type(keyi, jnp.float32)
    pad = float(jnp.finfo(jnp.float32).max)

    mp = jnp.float32(-1.0)
    idxs = []
    for _ in range(K_C):
        acc = None
        for c in range(0, P, CHUNK):
            v = key[c:c + CHUNK]
            v = jnp.where(v > mp, v, pad)
            n = CHUNK
            while n > 8:
                h = n // 2
                v = jnp.minimum(v[:h], v[h:])
                n = h
            acc = v if acc is None else jnp.minimum(acc, v)
        n = 8
        while n > 1:
            h = n // 2
            acc = jnp.minimum(acc[:h], acc[h:])
            n = h
        mp = acc
        idxs.append(lax.bitcast_convert_type(mp, jnp.int32)
                    & jnp.int32(P - 1))

    idx_ref[...] = jnp.concatenate(idxs, axis=0)


def _topk_call(qcols, pcols, Q, P, bQ):
    grid = (Q // bQ,)
    qspec = pl.BlockSpec((1, bQ), lambda i: (0, i))
    pspec = pl.BlockSpec((P, 1), lambda i: (0, 0))
    ospec = pl.BlockSpec((K_C, bQ), lambda i: (0, i))
    return pl.pallas_call(
        functools.partial(_topk_body, P=P),
        grid=grid,
        in_specs=[qspec] * 3 + [pspec] * 3,
        out_specs=[ospec],
        out_shape=[jax.ShapeDtypeStruct((K_C, Q), jnp.int32)],
    )(*qcols, *pcols)[0]


def _gather_call(idx_flat, pcols_flat, qreps, P):
    N = idx_flat.shape[0]
    info = plsc.get_sparse_core_info()
    NC, NS = info.num_cores, info.num_subcores
    NW = NC * NS
    n_w = N // NW
    mesh = plsc.VectorSubcoreMesh(core_axis_name="c", subcore_axis_name="s")

    @functools.partial(
        pl.kernel, mesh=mesh,
        compiler_params=pltpu.CompilerParams(needs_layout_passes=False),
        out_type=[jax.ShapeDtypeStruct((N,), jnp.float32)] * 3,
        scratch_types=[pltpu.VMEM((n_w,), jnp.int32)]
                      + [pltpu.VMEM((P,), jnp.float32)] * 3
                      + [pltpu.VMEM((n_w,), jnp.float32)] * 3
                      + [pltpu.VMEM((n_w,), jnp.float32)] * 3,
    )
    def gather_k(idx_hbm, px_hbm, py_hbm, pz_hbm, qx_hbm, qy_hbm, qz_hbm,
                 ox_hbm, oy_hbm, oz_hbm,
                 idx_v, px_v, py_v, pz_v,
                 qx_v, qy_v, qz_v, ox_v, oy_v, oz_v):
        wid = lax.axis_index("s") * NC + lax.axis_index("c")
        base = wid * n_w
        pltpu.sync_copy(idx_hbm.at[pl.ds(base, n_w)], idx_v)
        pltpu.sync_copy(px_hbm, px_v)
        pltpu.sync_copy(py_hbm, py_v)
        pltpu.sync_copy(pz_hbm, pz_v)
        pltpu.sync_copy(qx_hbm.at[pl.ds(base, n_w)], qx_v)
        pltpu.sync_copy(qy_hbm.at[pl.ds(base, n_w)], qy_v)
        pltpu.sync_copy(qz_hbm.at[pl.ds(base, n_w)], qz_v)

        def body(j, carry):
            o = j * 16
            sl = pl.ds(o, 16)
            iv = idx_v[sl]
            gx = plsc.load_gather(px_v, [iv])
            gy = plsc.load_gather(py_v, [iv])
            gz = plsc.load_gather(pz_v, [iv])
            rx = gx - qx_v[sl]
            ry = gy - qy_v[sl]
            rz = gz - qz_v[sl]
            e2 = rx * rx + ry * ry + rz * rz
            _, sx = plsc.sort_key_val(e2, rx)
            _, sy = plsc.sort_key_val(e2, ry)
            _, sz = plsc.sort_key_val(e2, rz)
            ox_v[sl] = sx
            oy_v[sl] = sy
            oz_v[sl] = sz
            return carry

        lax.fori_loop(0, n_w // 16, body, 0)
        pltpu.sync_copy(ox_v, ox_hbm.at[pl.ds(base, n_w)])
        pltpu.sync_copy(oy_v, oy_hbm.at[pl.ds(base, n_w)])
        pltpu.sync_copy(oz_v, oz_hbm.at[pl.ds(base, n_w)])

    return gather_k(idx_flat, *pcols_flat, *qreps)


def _angles_body(rx_ref, ry_ref, rz_ref,
                 rdx_ref, rdy_ref, rdz_ref, d_ref, az_ref, el_ref):
    rx = rx_ref[...]
    ry = ry_ref[...]
    rz = rz_ref[...]
    d = jnp.sqrt(rx * rx + ry * ry + rz * rz)
    d_ref[...] = d
    inv = 1.0 / jnp.maximum(d, 1e-12)
    rdx_ref[...] = rx * inv
    rdy_ref[...] = ry * inv
    rdz_ref[...] = rz * inv
    c = rz / (d + EPS_A)
    el = _acos(c)
    sinel = jnp.sqrt(jnp.maximum(1.0 - c * c, 0.0))
    az = _acos(rx / (d * sinel + EPS_A))
    az_ref[...] = jnp.where(ry < 0.0, 2.0 * math.pi - az, az)
    el_ref[...] = el


def _angles_call(rx, ry, rz, rows, cols, brows):
    grid = (rows // brows,)
    spec = pl.BlockSpec((brows, cols), lambda i: (i, 0))
    return pl.pallas_call(
        _angles_body,
        grid=grid,
        in_specs=[spec] * 3,
        out_specs=[spec] * 6,
        out_shape=[jax.ShapeDtypeStruct((rows, cols), jnp.float32)] * 6,
    )(rx, ry, rz)


def kernel(ray_o, light_probe_pos):
    Q = ray_o.shape[0]
    P = light_probe_pos.shape[0]

    qcols = [ray_o[:, i].reshape(1, Q) for i in range(3)]
    pcols = [light_probe_pos[:, i].reshape(P, 1) for i in range(3)]
    pcols_flat = [light_probe_pos[:, i].reshape(P) for i in range(3)]

    idx = _topk_call(qcols, pcols, Q, P, bQ=512)

    qreps = [jnp.repeat(ray_o[:, i], K_C) for i in range(3)]
    srx, sry, srz = _gather_call(idx.T.reshape(-1), pcols_flat, qreps, P)

    N = Q * K_C
    cols = 2048
    rows = N // cols
    outs = _angles_call(srx.reshape(rows, cols), sry.reshape(rows, cols),
                        srz.reshape(rows, cols), rows, cols, brows=rows // 8)

    chans = [o.reshape(Q, K_C) for o in outs]
    return jnp.stack(chans, axis=-1)
